# baseline (speedup 1.0000x reference)
import functools

import jax
import jax.numpy as jnp
import numpy as np
from jax import lax
from jax.experimental import pallas as pl
from jax.experimental.pallas import tpu as pltpu
import jax.experimental.pallas.tpu_sc as plsc

MULS = 16
DIM = 4 * MULS
ROW = 128
NC, NS = 2, 16
GW = 128


def _sc_mesh():
    return plsc.VectorSubcoreMesh(
        core_axis_name="core", subcore_axis_name="subcore",
        num_cores=NC, num_subcores=NS)


def _sc_gather(table, idx):
    e = idx.shape[0]
    idx2 = idx.reshape(1, e)

    @functools.partial(
        pl.kernel,
        out_type=jax.ShapeDtypeStruct((e, ROW), jnp.float32),
        mesh=_sc_mesh())
    def k(tab_hbm, i_hbm, o_hbm):
        def body(i_vmem, o_vmem):
            pltpu.sync_copy(tab_hbm.at[i_vmem.at[0]], o_vmem)

        pltpu.emit_pipeline(
            body,
            grid=(e // GW,),
            in_specs=[pl.BlockSpec((1, GW), lambda i: (0, i))],
            out_specs=[pl.BlockSpec((GW, ROW), lambda i: (i, 0))],
            core_axis_name=("core", "subcore"),
            dimension_semantics=(pltpu.PARALLEL,),
        )(i_hbm, o_hbm)

    return k(table, idx2)


def _sc_scatter_add(rows, idx, n):
    e = idx.shape[0]
    idx2 = idx.reshape(1, e)
    zrows = jnp.zeros((n, ROW), jnp.float32)
    chunk = -(-n // NS) & ~7
    chunk = chunk + 8 if chunk * NS < n else chunk
    rem = n - chunk * (NS - 1)
    assert 0 < rem <= chunk and chunk % 8 == 0

    @functools.partial(
        pl.kernel,
        out_type=jax.ShapeDtypeStruct((NC, n, ROW), jnp.float32),
        mesh=_sc_mesh(),
        scratch_types=[pltpu.VMEM_SHARED((n, ROW), jnp.float32)])
    def k(x_hbm, i_hbm, z_hbm, o_hbm, acc):
        cid = lax.axis_index("core")
        sid = lax.axis_index("subcore")

        @pl.when(sid < NS - 1)
        def _():
            sl = pl.ds(sid * chunk, chunk)
            pltpu.sync_copy(z_hbm.at[sl], acc.at[sl])

        @pl.when(sid == NS - 1)
        def _():
            sl = pl.ds((NS - 1) * chunk, rem)
            pltpu.sync_copy(z_hbm.at[sl], acc.at[sl])

        plsc.subcore_barrier()

        def body(x_vmem, i_vmem):
            pltpu.sync_copy(x_vmem, acc.at[i_vmem.at[0]], add=True)

        pltpu.emit_pipeline(
            body,
            grid=(e // GW,),
            in_specs=[pl.BlockSpec((GW, ROW), lambda i: (i, 0)),
                      pl.BlockSpec((1, GW), lambda i: (0, i))],
            core_axis_name=("core", "subcore"),
            dimension_semantics=(pltpu.PARALLEL,),
        )(x_hbm, i_hbm)
        plsc.subcore_barrier()

        @pl.when(sid < NS - 1)
        def _():
            sl = pl.ds(sid * chunk, chunk)
            pltpu.sync_copy(acc.at[sl], o_hbm.at[cid, sl])

        @pl.when(sid == NS - 1)
        def _():
            sl = pl.ds((NS - 1) * chunk, rem)
            pltpu.sync_copy(acc.at[sl], o_hbm.at[cid, sl])

    return k(rows, idx2, zrows)


def _tc_dense(xg, ea, ef, W1, b1, W2, b2, T, S, block_e):
    e = xg.shape[0]
    m = MULS
    inv = 1.0 / np.sqrt(2.0 * m)
    c3 = 1.0 / np.sqrt(3.0)
    scale = inv / 16.0
    bf = jnp.bfloat16
    f32 = jnp.float32

    def body(xg_ref, ea_ref, ef_ref, w1_ref, b1_ref, w2_ref, b2_ref,
             t_ref, s_ref, o_ref):
        mmf = functools.partial(jnp.dot, preferred_element_type=f32)

        def mmb(a, b):
            return mmf(a, b).astype(bf)
        ef_b = jnp.transpose(ef_ref[...])
        h = jnp.maximum(mmf(ef_b, w1_ref[...]) + b1_ref[...], 0.0)
        w = mmb(h.astype(bf), w2_ref[...].astype(bf)) + b2_ref[...].astype(bf)
        xg_b = xg_ref[...]
        ea_b = jnp.transpose(ea_ref[...])
        t = t_ref[...].astype(bf)
        s = s_ref[...].astype(bf)
        xs = xg_b[:, :m]
        xv0 = xg_b[:, m:2 * m]
        xv1 = xg_b[:, 2 * m:3 * m]
        xv2 = xg_b[:, 3 * m:4 * m]
        shs = ea_b[:, 0:1]
        sv0 = ea_b[:, 1:2]
        sv1 = ea_b[:, 2:3]
        sv2 = ea_b[:, 3:4]
        wp1 = w[:, :m * m]
        wp2 = w[:, m * m:2 * m * m]
        wp3 = w[:, 2 * m * m:3 * m * m]
        wp4 = w[:, 3 * m * m:4 * m * m]

        xvdot = xv0 * sv0 + xv1 * sv1 + xv2 * sv2
        a0 = mmb((xs * shs).astype(bf), t)
        a4 = mmb((xvdot * c3).astype(bf), t)
        outs = mmf(wp1 * a0 + wp4 * a4, s)
        ovs = []
        for sv, xv in ((sv0, xv0), (sv1, xv1), (sv2, xv2)):
            bk = mmb((xs * sv).astype(bf), t)
            ck = mmb((xv * shs).astype(bf), t)
            ovs.append(mmf(wp2 * bk + wp3 * ck, s))
        pad = jnp.zeros((xg_b.shape[0], ROW - DIM), f32)
        o_ref[...] = jnp.concatenate(
            [jnp.concatenate([outs] + ovs, axis=1) * scale, pad], axis=1)

    grid = (e // block_e,)
    return pl.pallas_call(
        body,
        grid=grid,
        in_specs=[
            pl.BlockSpec((block_e, ROW), lambda i: (i, 0)),
            pl.BlockSpec((4, block_e), lambda i: (0, i)),
            pl.BlockSpec((ef.shape[0], block_e), lambda i: (0, i)),
            pl.BlockSpec(W1.shape, lambda i: (0, 0)),
            pl.BlockSpec(b1.shape, lambda i: (0, 0)),
            pl.BlockSpec(W2.shape, lambda i: (0, 0)),
            pl.BlockSpec(b2.shape, lambda i: (0, 0)),
            pl.BlockSpec(T.shape, lambda i: (0, 0)),
            pl.BlockSpec(S.shape, lambda i: (0, 0)),
        ],
        out_specs=pl.BlockSpec((block_e, ROW), lambda i: (i, 0)),
        out_shape=jax.ShapeDtypeStruct((e, ROW), jnp.float32),
    )(xg, ea, ef, W1, b1, W2, b2, T, S)


def _tc_combine(parts, P):
    n = parts.shape[1]

    def body(p_ref, perm_ref, o_ref):
        o_ref[...] = jnp.dot(p_ref[0] + p_ref[1], perm_ref[...],
                             preferred_element_type=jnp.float32)

    return pl.pallas_call(
        body,
        out_shape=jax.ShapeDtypeStruct((n, DIM), jnp.float32),
    )(parts, P)


def kernel(node_attr, edge_index, edge_attr, edge_feat, W1, b1, W2, b2):
    n, m = node_attr.shape[0], MULS
    src = edge_index[0]
    dst = edge_index[1]

    na = jnp.concatenate(
        [node_attr[:, :m], node_attr[:, m + 0::3], node_attr[:, m + 1::3],
         node_attr[:, m + 2::3],
         jnp.zeros((n, ROW - DIM), node_attr.dtype)], axis=1)

    T = jnp.asarray(np.kron(np.eye(m), np.ones((1, m))), jnp.float32)
    S = jnp.asarray(np.kron(np.ones((m, 1)), np.eye(m)), jnp.float32)
    Pm = np.zeros((ROW, DIM), np.float32)
    for u in range(m):
        Pm[u, u] = 1.0
    for k in range(3):
        for w_ in range(m):
            Pm[m + m * k + w_, m + 3 * w_ + k] = 1.0
    P = jnp.asarray(Pm)

    xg = _sc_gather(na, dst)
    tp = _tc_dense(xg, edge_attr.T, edge_feat.T, W1, b1.reshape(1, -1),
                   W2, b2.reshape(1, -1), T, S, block_e=640)
    parts = _sc_scatter_add(tp, src, n)
    return _tc_combine(parts, P)


# speedup vs baseline: 1.3138x; 1.3138x over previous
import functools

import jax
import jax.numpy as jnp
import numpy as np
from jax import lax
from jax.experimental import pallas as pl
from jax.experimental.pallas import tpu as pltpu
import jax.experimental.pallas.tpu_sc as plsc

MULS = 16
DIM = 4 * MULS
ROW = 128
NC, NS = 2, 16
GW = 128


def _sc_mesh():
    return plsc.VectorSubcoreMesh(
        core_axis_name="core", subcore_axis_name="subcore",
        num_cores=NC, num_subcores=NS)


def _sc_gather(table, idx):
    e = idx.shape[0]
    idx2 = idx.reshape(1, e)

    @functools.partial(
        pl.kernel,
        out_type=jax.ShapeDtypeStruct((e, ROW), jnp.float32),
        mesh=_sc_mesh())
    def k(tab_hbm, i_hbm, o_hbm):
        def body(i_vmem, o_vmem):
            pltpu.sync_copy(tab_hbm.at[i_vmem.at[0]], o_vmem)

        pltpu.emit_pipeline(
            body,
            grid=(e // GW,),
            in_specs=[pl.BlockSpec((1, GW), lambda i: (0, i))],
            out_specs=[pl.BlockSpec((GW, ROW), lambda i: (i, 0))],
            core_axis_name=("core", "subcore"),
            dimension_semantics=(pltpu.PARALLEL,),
        )(i_hbm, o_hbm)

    return k(table, idx2)


def _sc_scatter_add(rows, idx, n):
    e = idx.shape[0]
    idx2 = idx.reshape(1, e)
    zrows = jnp.zeros((n, ROW), jnp.float32)
    chunk = -(-n // NS) & ~7
    chunk = chunk + 8 if chunk * NS < n else chunk
    rem = n - chunk * (NS - 1)
    assert 0 < rem <= chunk and chunk % 8 == 0

    @functools.partial(
        pl.kernel,
        out_type=jax.ShapeDtypeStruct((NC, n, ROW), jnp.float32),
        mesh=_sc_mesh(),
        scratch_types=[pltpu.VMEM_SHARED((n, ROW), jnp.float32)])
    def k(x_hbm, i_hbm, z_hbm, o_hbm, acc):
        cid = lax.axis_index("core")
        sid = lax.axis_index("subcore")

        @pl.when(sid < NS - 1)
        def _():
            sl = pl.ds(sid * chunk, chunk)
            pltpu.sync_copy(z_hbm.at[sl], acc.at[sl])

        @pl.when(sid == NS - 1)
        def _():
            sl = pl.ds((NS - 1) * chunk, rem)
            pltpu.sync_copy(z_hbm.at[sl], acc.at[sl])

        plsc.subcore_barrier()

        def body(x_vmem, i_vmem):
            pltpu.sync_copy(x_vmem, acc.at[i_vmem.at[0]], add=True)

        pltpu.emit_pipeline(
            body,
            grid=(e // GW,),
            in_specs=[pl.BlockSpec((GW, ROW), lambda i: (i, 0)),
                      pl.BlockSpec((1, GW), lambda i: (0, i))],
            core_axis_name=("core", "subcore"),
            dimension_semantics=(pltpu.PARALLEL,),
        )(x_hbm, i_hbm)
        plsc.subcore_barrier()

        @pl.when(sid < NS - 1)
        def _():
            sl = pl.ds(sid * chunk, chunk)
            pltpu.sync_copy(acc.at[sl], o_hbm.at[cid, sl])

        @pl.when(sid == NS - 1)
        def _():
            sl = pl.ds((NS - 1) * chunk, rem)
            pltpu.sync_copy(acc.at[sl], o_hbm.at[cid, sl])

    return k(rows, idx2, zrows)


def _tc_dense(xg, ea, ef, W1, b1, W2, b2, T, S, block_e):
    e = xg.shape[0]
    m = MULS
    bf = jnp.bfloat16
    f32 = jnp.float32

    def body(xg_ref, ea_ref, ef_ref, w1_ref, b1_ref, w2_ref, b2_ref,
             t_ref, s_ref, o_ref):
        mmf = functools.partial(jnp.dot, preferred_element_type=f32)

        def mmb(a, b):
            return mmf(a, b).astype(bf)
        ef_b = jnp.transpose(ef_ref[...])
        h = jnp.maximum(mmf(ef_b, w1_ref[...]) + b1_ref[...], 0.0)
        w = mmb(h.astype(bf), w2_ref[...].astype(bf)) + b2_ref[...].astype(bf)
        xg_b = xg_ref[...].astype(bf)
        ea_b = jnp.transpose(ea_ref[...]).astype(bf)
        t = t_ref[...].astype(bf)
        s = s_ref[...].astype(bf)
        shs = ea_b[:, 0:1]
        sv0 = ea_b[:, 1:2]
        sv1 = ea_b[:, 2:3]
        sv2 = ea_b[:, 3:4]
        wp1 = w[:, :m * m]
        wp2 = w[:, m * m:2 * m * m]
        wp3 = w[:, 2 * m * m:3 * m * m]
        wp4 = w[:, 3 * m * m:4 * m * m]

        xt = mmb(xg_b[:, :DIM], t)
        xst = xt[:, :m * m]
        xv0t = xt[:, m * m:2 * m * m]
        xv1t = xt[:, 2 * m * m:3 * m * m]
        xv2t = xt[:, 3 * m * m:4 * m * m]
        a4 = xv0t * sv0 + xv1t * sv1 + xv2t * sv2
        p2 = wp2 * xst
        outs = mmf(wp1 * xst * shs + wp4 * a4, s)
        ov0 = mmf(p2 * sv0 + wp3 * xv0t * shs, s)
        ov1 = mmf(p2 * sv1 + wp3 * xv1t * shs, s)
        ov2 = mmf(p2 * sv2 + wp3 * xv2t * shs, s)
        pad = jnp.zeros((xg_b.shape[0], ROW - DIM), f32)
        o_ref[...] = jnp.concatenate([outs, ov0, ov1, ov2, pad], axis=1)

    grid = (e // block_e,)
    return pl.pallas_call(
        body,
        grid=grid,
        in_specs=[
            pl.BlockSpec((block_e, ROW), lambda i: (i, 0)),
            pl.BlockSpec((4, block_e), lambda i: (0, i)),
            pl.BlockSpec((ef.shape[0], block_e), lambda i: (0, i)),
            pl.BlockSpec(W1.shape, lambda i: (0, 0)),
            pl.BlockSpec(b1.shape, lambda i: (0, 0)),
            pl.BlockSpec(W2.shape, lambda i: (0, 0)),
            pl.BlockSpec(b2.shape, lambda i: (0, 0)),
            pl.BlockSpec(T.shape, lambda i: (0, 0)),
            pl.BlockSpec(S.shape, lambda i: (0, 0)),
        ],
        out_specs=pl.BlockSpec((block_e, ROW), lambda i: (i, 0)),
        out_shape=jax.ShapeDtypeStruct((e, ROW), jnp.float32),
    )(xg, ea, ef, W1, b1, W2, b2, T, S)


def _tc_combine(parts, P):
    n = parts.shape[1]

    def body(p_ref, perm_ref, o_ref):
        o_ref[...] = jnp.dot(p_ref[0] + p_ref[1], perm_ref[...],
                             preferred_element_type=jnp.float32)

    return pl.pallas_call(
        body,
        out_shape=jax.ShapeDtypeStruct((n, DIM), jnp.float32),
    )(parts, P)


def kernel(node_attr, edge_index, edge_attr, edge_feat, W1, b1, W2, b2):
    n, m = node_attr.shape[0], MULS
    src = edge_index[0]
    dst = edge_index[1]

    na = jnp.concatenate(
        [node_attr[:, :m], node_attr[:, m + 0::3], node_attr[:, m + 1::3],
         node_attr[:, m + 2::3],
         jnp.zeros((n, ROW - DIM), node_attr.dtype)], axis=1)

    T = jnp.asarray(np.kron(np.eye(DIM), np.ones((1, m))), jnp.float32)
    S = jnp.asarray(np.kron(np.ones((m, 1)), np.eye(m)), jnp.float32)
    cvec = np.full((4 * m * m,), 1.0 / (np.sqrt(2.0 * m) * 16.0), np.float32)
    cvec[3 * m * m:] /= np.sqrt(3.0)
    cvec = jnp.asarray(cvec)
    W2 = W2 * cvec
    b2 = b2 * cvec
    Pm = np.zeros((ROW, DIM), np.float32)
    for u in range(m):
        Pm[u, u] = 1.0
    for k in range(3):
        for w_ in range(m):
            Pm[m + m * k + w_, m + 3 * w_ + k] = 1.0
    P = jnp.asarray(Pm)

    xg = _sc_gather(na, dst)
    tp = _tc_dense(xg, edge_attr.T, edge_feat.T, W1, b1.reshape(1, -1),
                   W2, b2.reshape(1, -1), T, S, block_e=640)
    parts = _sc_scatter_add(tp, src, n)
    return _tc_combine(parts, P)


# speedup vs baseline: 1.4835x; 1.1292x over previous
import functools

import jax
import jax.numpy as jnp
import numpy as np
from jax import lax
from jax.experimental import pallas as pl
from jax.experimental.pallas import tpu as pltpu
import jax.experimental.pallas.tpu_sc as plsc

MULS = 16
DIM = 4 * MULS
ROW = 128
NC, NS = 2, 16
GW = 128


def _sc_mesh():
    return plsc.VectorSubcoreMesh(
        core_axis_name="core", subcore_axis_name="subcore",
        num_cores=NC, num_subcores=NS)


def _sc_gather(table, idx):
    e = idx.shape[0]
    idx2 = idx.reshape(1, e)

    @functools.partial(
        pl.kernel,
        out_type=jax.ShapeDtypeStruct((e, ROW), jnp.float32),
        mesh=_sc_mesh())
    def k(tab_hbm, i_hbm, o_hbm):
        def body(i_vmem, o_vmem):
            pltpu.sync_copy(tab_hbm.at[i_vmem.at[0]], o_vmem)

        pltpu.emit_pipeline(
            body,
            grid=(e // GW,),
            in_specs=[pl.BlockSpec((1, GW), lambda i: (0, i))],
            out_specs=[pl.BlockSpec((GW, ROW), lambda i: (i, 0))],
            core_axis_name=("core", "subcore"),
            dimension_semantics=(pltpu.PARALLEL,),
        )(i_hbm, o_hbm)

    return k(table, idx2)


def _sc_scatter_add(rows, idx, n):
    e = idx.shape[0]
    idx2 = idx.reshape(1, e)
    zrows = jnp.zeros((n, ROW), jnp.float32)
    chunk = -(-n // NS) & ~7
    chunk = chunk + 8 if chunk * NS < n else chunk
    rem = n - chunk * (NS - 1)
    assert 0 < rem <= chunk and chunk % 8 == 0

    @functools.partial(
        pl.kernel,
        out_type=jax.ShapeDtypeStruct((NC, n, ROW), jnp.float32),
        mesh=_sc_mesh(),
        scratch_types=[pltpu.VMEM_SHARED((n, ROW), jnp.float32)])
    def k(x_hbm, i_hbm, z_hbm, o_hbm, acc):
        cid = lax.axis_index("core")
        sid = lax.axis_index("subcore")

        @pl.when(sid < NS - 1)
        def _():
            sl = pl.ds(sid * chunk, chunk)
            pltpu.sync_copy(z_hbm.at[sl], acc.at[sl])

        @pl.when(sid == NS - 1)
        def _():
            sl = pl.ds((NS - 1) * chunk, rem)
            pltpu.sync_copy(z_hbm.at[sl], acc.at[sl])

        plsc.subcore_barrier()

        def body(x_vmem, i_vmem):
            pltpu.sync_copy(x_vmem, acc.at[i_vmem.at[0]], add=True)

        pltpu.emit_pipeline(
            body,
            grid=(e // GW,),
            in_specs=[pl.BlockSpec((GW, ROW), lambda i: (i, 0)),
                      pl.BlockSpec((1, GW), lambda i: (0, i))],
            core_axis_name=("core", "subcore"),
            dimension_semantics=(pltpu.PARALLEL,),
        )(x_hbm, i_hbm)
        plsc.subcore_barrier()

        @pl.when(sid < NS - 1)
        def _():
            sl = pl.ds(sid * chunk, chunk)
            pltpu.sync_copy(acc.at[sl], o_hbm.at[cid, sl])

        @pl.when(sid == NS - 1)
        def _():
            sl = pl.ds((NS - 1) * chunk, rem)
            pltpu.sync_copy(acc.at[sl], o_hbm.at[cid, sl])

    return k(rows, idx2, zrows)


def _tc_dense(xg, ea, ef, W1, b1, W2, b2, T, S, block_e):
    e = xg.shape[0]
    m = MULS
    bf = jnp.bfloat16
    f32 = jnp.float32

    def body(xg_ref, ea_ref, ef_ref, w1_ref, b1_ref, w2_ref, b2_ref,
             t_ref, s_ref, o_ref):
        mmf = functools.partial(jnp.dot, preferred_element_type=f32)

        def mmb(a, b):
            return mmf(a, b).astype(bf)
        ef_b = jnp.transpose(ef_ref[...])
        h = jnp.maximum(mmf(ef_b, w1_ref[...]) + b1_ref[...], 0.0)
        w = mmb(h.astype(bf), w2_ref[...].astype(bf)) + b2_ref[...].astype(bf)
        xg_b = xg_ref[...].astype(bf)
        ea_b = jnp.transpose(ea_ref[...]).astype(bf)
        t = t_ref[...].astype(bf)
        s = s_ref[...].astype(bf)
        shs = ea_b[:, 0:1]
        sv0 = ea_b[:, 1:2]
        sv1 = ea_b[:, 2:3]
        sv2 = ea_b[:, 3:4]
        wp1 = w[:, :m * m]
        wp2 = w[:, m * m:2 * m * m]
        wp3 = w[:, 2 * m * m:3 * m * m]
        wp4 = w[:, 3 * m * m:4 * m * m]

        xt = mmb(xg_b[:, :DIM], t)
        xst = xt[:, :m * m]
        xv0t = xt[:, m * m:2 * m * m]
        xv1t = xt[:, 2 * m * m:3 * m * m]
        xv2t = xt[:, 3 * m * m:4 * m * m]
        a4 = xv0t * sv0 + xv1t * sv1 + xv2t * sv2
        p2 = wp2 * xst
        outs = mmf(wp1 * xst * shs + wp4 * a4, s)
        ov0 = mmf(p2 * sv0 + wp3 * xv0t * shs, s)
        ov1 = mmf(p2 * sv1 + wp3 * xv1t * shs, s)
        ov2 = mmf(p2 * sv2 + wp3 * xv2t * shs, s)
        pad = jnp.zeros((xg_b.shape[0], ROW - DIM), f32)
        o_ref[...] = jnp.concatenate([outs, ov0, ov1, ov2, pad], axis=1)

    grid = (e // block_e,)
    return pl.pallas_call(
        body,
        grid=grid,
        in_specs=[
            pl.BlockSpec((block_e, ROW), lambda i: (i, 0)),
            pl.BlockSpec((4, block_e), lambda i: (0, i)),
            pl.BlockSpec((ef.shape[0], block_e), lambda i: (0, i)),
            pl.BlockSpec(W1.shape, lambda i: (0, 0)),
            pl.BlockSpec(b1.shape, lambda i: (0, 0)),
            pl.BlockSpec(W2.shape, lambda i: (0, 0)),
            pl.BlockSpec(b2.shape, lambda i: (0, 0)),
            pl.BlockSpec(T.shape, lambda i: (0, 0)),
            pl.BlockSpec(S.shape, lambda i: (0, 0)),
        ],
        out_specs=pl.BlockSpec((block_e, ROW), lambda i: (i, 0)),
        out_shape=jax.ShapeDtypeStruct((e, ROW), jnp.float32),
    )(xg, ea, ef, W1, b1, W2, b2, T, S)


def _tc_combine(parts, P):
    n = parts.shape[1]

    def body(p_ref, perm_ref, o_ref):
        o_ref[...] = jnp.dot(p_ref[0] + p_ref[1], perm_ref[...],
                             preferred_element_type=jnp.float32)

    return pl.pallas_call(
        body,
        out_shape=jax.ShapeDtypeStruct((n, DIM), jnp.float32),
    )(parts, P)


def kernel(node_attr, edge_index, edge_attr, edge_feat, W1, b1, W2, b2):
    n, m = node_attr.shape[0], MULS
    src = edge_index[0]
    dst = edge_index[1]

    na = jnp.concatenate(
        [node_attr[:, :m], node_attr[:, m + 0::3], node_attr[:, m + 1::3],
         node_attr[:, m + 2::3],
         jnp.zeros((n, ROW - DIM), node_attr.dtype)], axis=1)

    T = jnp.asarray(np.kron(np.eye(DIM), np.ones((1, m))), jnp.float32)
    S = jnp.asarray(np.kron(np.ones((m, 1)), np.eye(m)), jnp.float32)
    cvec = np.full((4 * m * m,), 1.0 / (np.sqrt(2.0 * m) * 16.0), np.float32)
    cvec[3 * m * m:] /= np.sqrt(3.0)
    cvec = jnp.asarray(cvec)
    W2 = W2 * cvec
    b2 = b2 * cvec
    Pm = np.zeros((ROW, DIM), np.float32)
    for u in range(m):
        Pm[u, u] = 1.0
    for k in range(3):
        for w_ in range(m):
            Pm[m + m * k + w_, m + 3 * w_ + k] = 1.0
    P = jnp.asarray(Pm)

    xg = _sc_gather(na, dst)
    tp = _tc_dense(xg, edge_attr.T, edge_feat.T, W1, b1.reshape(1, -1),
                   W2, b2.reshape(1, -1), T, S, block_e=1280)
    parts = _sc_scatter_add(tp, src, n)
    return _tc_combine(parts, P)


# speedup vs baseline: 1.5433x; 1.0404x over previous
import functools

import jax
import jax.numpy as jnp
import numpy as np
from jax import lax
from jax.experimental import pallas as pl
from jax.experimental.pallas import tpu as pltpu
import jax.experimental.pallas.tpu_sc as plsc

MULS = 16
DIM = 4 * MULS
ROW = 128
NC, NS = 2, 16
GW = 128


def _sc_mesh():
    return plsc.VectorSubcoreMesh(
        core_axis_name="core", subcore_axis_name="subcore",
        num_cores=NC, num_subcores=NS)


def _sc_gather(table, idx):
    e = idx.shape[0]
    idx2 = idx.reshape(1, e)

    @functools.partial(
        pl.kernel,
        out_type=jax.ShapeDtypeStruct((e, ROW), jnp.float32),
        mesh=_sc_mesh())
    def k(tab_hbm, i_hbm, o_hbm):
        def body(i_vmem, o_vmem):
            pltpu.sync_copy(tab_hbm.at[i_vmem.at[0]], o_vmem)

        pltpu.emit_pipeline(
            body,
            grid=(e // GW,),
            in_specs=[pl.BlockSpec((1, GW), lambda i: (0, i))],
            out_specs=[pl.BlockSpec((GW, ROW), lambda i: (i, 0))],
            core_axis_name=("core", "subcore"),
            dimension_semantics=(pltpu.PARALLEL,),
        )(i_hbm, o_hbm)

    return k(table, idx2)


def _sc_scatter_add(rows, idx, n):
    e = idx.shape[0]
    idx2 = idx.reshape(1, e)
    zrows = jnp.zeros((n, ROW), jnp.float32)
    chunk = -(-n // NS) & ~7
    chunk = chunk + 8 if chunk * NS < n else chunk
    rem = n - chunk * (NS - 1)
    assert 0 < rem <= chunk and chunk % 8 == 0

    @functools.partial(
        pl.kernel,
        out_type=jax.ShapeDtypeStruct((NC, n, ROW), jnp.float32),
        mesh=_sc_mesh(),
        scratch_types=[pltpu.VMEM_SHARED((n, ROW), jnp.float32)])
    def k(x_hbm, i_hbm, z_hbm, o_hbm, acc):
        cid = lax.axis_index("core")
        sid = lax.axis_index("subcore")

        @pl.when(sid < NS - 1)
        def _():
            sl = pl.ds(sid * chunk, chunk)
            pltpu.sync_copy(z_hbm.at[sl], acc.at[sl])

        @pl.when(sid == NS - 1)
        def _():
            sl = pl.ds((NS - 1) * chunk, rem)
            pltpu.sync_copy(z_hbm.at[sl], acc.at[sl])

        plsc.subcore_barrier()

        def body(x_vmem, i_vmem):
            pltpu.sync_copy(x_vmem, acc.at[i_vmem.at[0]], add=True)

        pltpu.emit_pipeline(
            body,
            grid=(e // GW,),
            in_specs=[pl.BlockSpec((GW, ROW), lambda i: (i, 0)),
                      pl.BlockSpec((1, GW), lambda i: (0, i))],
            core_axis_name=("core", "subcore"),
            dimension_semantics=(pltpu.PARALLEL,),
        )(x_hbm, i_hbm)
        plsc.subcore_barrier()

        @pl.when(sid < NS - 1)
        def _():
            sl = pl.ds(sid * chunk, chunk)
            pltpu.sync_copy(acc.at[sl], o_hbm.at[cid, sl])

        @pl.when(sid == NS - 1)
        def _():
            sl = pl.ds((NS - 1) * chunk, rem)
            pltpu.sync_copy(acc.at[sl], o_hbm.at[cid, sl])

    return k(rows, idx2, zrows)


def _tc_dense(xg, ea, ef, W1, b1, W2, b2, T, S, block_e):
    e = xg.shape[0]
    m = MULS
    bf = jnp.bfloat16
    f32 = jnp.float32

    def body(xg_ref, ea_ref, ef_ref, w1_ref, b1_ref, w2_ref, b2_ref,
             t_ref, s_ref, o_ref):
        mmf = functools.partial(jnp.dot, preferred_element_type=f32)

        def mmb(a, b):
            return mmf(a, b).astype(bf)
        ef_b = jnp.transpose(ef_ref[...])
        h = jnp.maximum(mmf(ef_b, w1_ref[...]) + b1_ref[...], 0.0)
        w = mmb(h.astype(bf), w2_ref[...].astype(bf)) + b2_ref[...].astype(bf)
        xg_b = xg_ref[...].astype(bf)
        ea_b = jnp.transpose(ea_ref[...]).astype(bf)
        t = t_ref[...].astype(bf)
        s = s_ref[...].astype(bf)
        shs = ea_b[:, 0:1]
        sv0 = ea_b[:, 1:2]
        sv1 = ea_b[:, 2:3]
        sv2 = ea_b[:, 3:4]
        wp1 = w[:, :m * m]
        wp2 = w[:, m * m:2 * m * m]
        wp3 = w[:, 2 * m * m:3 * m * m]
        wp4 = w[:, 3 * m * m:4 * m * m]

        xt = mmb(xg_b[:, :DIM], t)
        xst = xt[:, :m * m]
        xv0t = xt[:, m * m:2 * m * m]
        xv1t = xt[:, 2 * m * m:3 * m * m]
        xv2t = xt[:, 3 * m * m:4 * m * m]
        a4 = xv0t * sv0 + xv1t * sv1 + xv2t * sv2
        p2 = wp2 * xst
        outs = mmf(wp1 * xst * shs + wp4 * a4, s)
        ov0 = mmf(p2 * sv0 + wp3 * xv0t * shs, s)
        ov1 = mmf(p2 * sv1 + wp3 * xv1t * shs, s)
        ov2 = mmf(p2 * sv2 + wp3 * xv2t * shs, s)
        pad = jnp.zeros((xg_b.shape[0], ROW - DIM), f32)
        o_ref[...] = jnp.concatenate([outs, ov0, ov1, ov2, pad], axis=1)

    grid = (e // block_e,)
    return pl.pallas_call(
        body,
        grid=grid,
        in_specs=[
            pl.BlockSpec((block_e, ROW), lambda i: (i, 0)),
            pl.BlockSpec((4, block_e), lambda i: (0, i)),
            pl.BlockSpec((ef.shape[0], block_e), lambda i: (0, i)),
            pl.BlockSpec(W1.shape, lambda i: (0, 0)),
            pl.BlockSpec(b1.shape, lambda i: (0, 0)),
            pl.BlockSpec(W2.shape, lambda i: (0, 0)),
            pl.BlockSpec(b2.shape, lambda i: (0, 0)),
            pl.BlockSpec(T.shape, lambda i: (0, 0)),
            pl.BlockSpec(S.shape, lambda i: (0, 0)),
        ],
        out_specs=pl.BlockSpec((block_e, ROW), lambda i: (i, 0)),
        out_shape=jax.ShapeDtypeStruct((e, ROW), jnp.float32),
    )(xg, ea, ef, W1, b1, W2, b2, T, S)


def _tc_combine(parts, P):
    n = parts.shape[1]

    def body(p_ref, perm_ref, o_ref):
        o_ref[...] = jnp.dot(p_ref[0] + p_ref[1], perm_ref[...],
                             preferred_element_type=jnp.float32)

    return pl.pallas_call(
        body,
        out_shape=jax.ShapeDtypeStruct((n, DIM), jnp.float32),
    )(parts, P)


def kernel(node_attr, edge_index, edge_attr, edge_feat, W1, b1, W2, b2):
    n, m = node_attr.shape[0], MULS
    src = edge_index[0]
    dst = edge_index[1]

    na = jnp.concatenate(
        [node_attr[:, :m], node_attr[:, m + 0::3], node_attr[:, m + 1::3],
         node_attr[:, m + 2::3],
         jnp.zeros((n, ROW - DIM), node_attr.dtype)], axis=1)

    T = jnp.asarray(np.kron(np.eye(DIM), np.ones((1, m))), jnp.float32)
    S = jnp.asarray(np.kron(np.ones((m, 1)), np.eye(m)), jnp.float32)
    cvec = np.full((4 * m * m,), 1.0 / (np.sqrt(2.0 * m) * 16.0), np.float32)
    cvec[3 * m * m:] /= np.sqrt(3.0)
    cvec = jnp.asarray(cvec)
    W2 = W2 * cvec
    b2 = b2 * cvec
    Pm = np.zeros((ROW, DIM), np.float32)
    for u in range(m):
        Pm[u, u] = 1.0
    for k in range(3):
        for w_ in range(m):
            Pm[m + m * k + w_, m + 3 * w_ + k] = 1.0
    P = jnp.asarray(Pm)

    xg = _sc_gather(na, dst)
    tp = _tc_dense(xg, edge_attr.T, edge_feat.T, W1, b1.reshape(1, -1),
                   W2, b2.reshape(1, -1), T, S, block_e=3200)
    parts = _sc_scatter_add(tp, src, n)
    return _tc_combine(parts, P)


# speedup vs baseline: 1.5672x; 1.0154x over previous
import functools

import jax
import jax.numpy as jnp
import numpy as np
from jax import lax
from jax.experimental import pallas as pl
from jax.experimental.pallas import tpu as pltpu
import jax.experimental.pallas.tpu_sc as plsc

MULS = 16
DIM = 4 * MULS
ROW = 128
NC, NS = 2, 16
GW = 128


def _sc_mesh():
    return plsc.VectorSubcoreMesh(
        core_axis_name="core", subcore_axis_name="subcore",
        num_cores=NC, num_subcores=NS)


def _sc_gather(table, idx):
    e = idx.shape[0]
    idx2 = idx.reshape(1, e)

    @functools.partial(
        pl.kernel,
        out_type=jax.ShapeDtypeStruct((e, ROW), jnp.float32),
        mesh=_sc_mesh())
    def k(tab_hbm, i_hbm, o_hbm):
        def body(i_vmem, o_vmem):
            pltpu.sync_copy(tab_hbm.at[i_vmem.at[0]], o_vmem)

        pltpu.emit_pipeline(
            body,
            grid=(e // GW,),
            in_specs=[pl.BlockSpec((1, GW), lambda i: (0, i))],
            out_specs=[pl.BlockSpec((GW, ROW), lambda i: (i, 0))],
            core_axis_name=("core", "subcore"),
            dimension_semantics=(pltpu.PARALLEL,),
        )(i_hbm, o_hbm)

    return k(table, idx2)


def _sc_scatter_add(rows, idx, n):
    e = idx.shape[0]
    idx2 = idx.reshape(1, e)
    zrows = jnp.zeros((n, ROW), jnp.float32)
    chunk = -(-n // NS) & ~7
    chunk = chunk + 8 if chunk * NS < n else chunk
    rem = n - chunk * (NS - 1)
    assert 0 < rem <= chunk and chunk % 8 == 0

    @functools.partial(
        pl.kernel,
        out_type=jax.ShapeDtypeStruct((NC, n, ROW), jnp.float32),
        mesh=_sc_mesh(),
        scratch_types=[pltpu.VMEM_SHARED((n, ROW), jnp.float32)])
    def k(x_hbm, i_hbm, z_hbm, o_hbm, acc):
        cid = lax.axis_index("core")
        sid = lax.axis_index("subcore")

        @pl.when(sid < NS - 1)
        def _():
            sl = pl.ds(sid * chunk, chunk)
            pltpu.sync_copy(z_hbm.at[sl], acc.at[sl])

        @pl.when(sid == NS - 1)
        def _():
            sl = pl.ds((NS - 1) * chunk, rem)
            pltpu.sync_copy(z_hbm.at[sl], acc.at[sl])

        plsc.subcore_barrier()

        def body(x_vmem, i_vmem):
            pltpu.sync_copy(x_vmem, acc.at[i_vmem.at[0]], add=True)

        pltpu.emit_pipeline(
            body,
            grid=(e // GW,),
            in_specs=[pl.BlockSpec((GW, ROW), lambda i: (i, 0)),
                      pl.BlockSpec((1, GW), lambda i: (0, i))],
            core_axis_name=("core", "subcore"),
            dimension_semantics=(pltpu.PARALLEL,),
        )(x_hbm, i_hbm)
        plsc.subcore_barrier()

        @pl.when(sid < NS - 1)
        def _():
            sl = pl.ds(sid * chunk, chunk)
            pltpu.sync_copy(acc.at[sl], o_hbm.at[cid, sl])

        @pl.when(sid == NS - 1)
        def _():
            sl = pl.ds((NS - 1) * chunk, rem)
            pltpu.sync_copy(acc.at[sl], o_hbm.at[cid, sl])

    return k(rows, idx2, zrows)


def _tc_dense(xg, ea, ef, W1, b1, W2, b2, T, S, block_e):
    e = xg.shape[0]
    m = MULS
    bf = jnp.bfloat16
    f32 = jnp.float32

    def body(xg_ref, ea_ref, ef_ref, w1_ref, b1_ref, w2_ref, b2_ref,
             t_ref, s_ref, o_ref):
        mmf = functools.partial(jnp.dot, preferred_element_type=f32)

        def mmb(a, b):
            return mmf(a, b).astype(bf)
        ef_b = jnp.transpose(ef_ref[...])
        h = jnp.maximum(mmf(ef_b, w1_ref[...]) + b1_ref[...], 0.0)
        w = mmb(h.astype(bf), w2_ref[...].astype(bf)) + b2_ref[...].astype(bf)
        xg_b = xg_ref[...].astype(bf)
        ea_b = jnp.transpose(ea_ref[...]).astype(bf)
        t = t_ref[...].astype(bf)
        s = s_ref[...].astype(bf)
        shs = ea_b[:, 0:1]
        sv0 = ea_b[:, 1:2]
        sv1 = ea_b[:, 2:3]
        sv2 = ea_b[:, 3:4]
        wp1 = w[:, :m * m]
        wp2 = w[:, m * m:2 * m * m]
        wp3 = w[:, 2 * m * m:3 * m * m]
        wp4 = w[:, 3 * m * m:4 * m * m]

        xt = mmb(xg_b[:, :DIM], t)
        xst = xt[:, :m * m]
        xv0t = xt[:, m * m:2 * m * m]
        xv1t = xt[:, 2 * m * m:3 * m * m]
        xv2t = xt[:, 3 * m * m:4 * m * m]
        a4 = xv0t * sv0 + xv1t * sv1 + xv2t * sv2
        p2 = wp2 * xst
        outs = mmf(wp1 * xst * shs + wp4 * a4, s)
        ov0 = mmf(p2 * sv0 + wp3 * xv0t * shs, s)
        ov1 = mmf(p2 * sv1 + wp3 * xv1t * shs, s)
        ov2 = mmf(p2 * sv2 + wp3 * xv2t * shs, s)
        pad = jnp.zeros((xg_b.shape[0], ROW - DIM), f32)
        o_ref[...] = jnp.concatenate([outs, ov0, ov1, ov2, pad], axis=1)

    grid = (e // block_e,)
    return pl.pallas_call(
        body,
        grid=grid,
        in_specs=[
            pl.BlockSpec((block_e, ROW), lambda i: (i, 0)),
            pl.BlockSpec((4, block_e), lambda i: (0, i)),
            pl.BlockSpec((ef.shape[0], block_e), lambda i: (0, i)),
            pl.BlockSpec(W1.shape, lambda i: (0, 0)),
            pl.BlockSpec(b1.shape, lambda i: (0, 0)),
            pl.BlockSpec(W2.shape, lambda i: (0, 0)),
            pl.BlockSpec(b2.shape, lambda i: (0, 0)),
            pl.BlockSpec(T.shape, lambda i: (0, 0)),
            pl.BlockSpec(S.shape, lambda i: (0, 0)),
        ],
        out_specs=pl.BlockSpec((block_e, ROW), lambda i: (i, 0)),
        out_shape=jax.ShapeDtypeStruct((e, ROW), jnp.float32),
    )(xg, ea, ef, W1, b1, W2, b2, T, S)


def _tc_combine(parts, P):
    n = parts.shape[1]

    def body(p_ref, perm_ref, o_ref):
        o_ref[...] = jnp.dot(p_ref[0] + p_ref[1], perm_ref[...],
                             preferred_element_type=jnp.float32)

    return pl.pallas_call(
        body,
        out_shape=jax.ShapeDtypeStruct((n, DIM), jnp.float32),
    )(parts, P)


def kernel(node_attr, edge_index, edge_attr, edge_feat, W1, b1, W2, b2):
    n, m = node_attr.shape[0], MULS
    src = edge_index[0]
    dst = edge_index[1]

    na = jnp.concatenate(
        [node_attr[:, :m], node_attr[:, m + 0::3], node_attr[:, m + 1::3],
         node_attr[:, m + 2::3],
         jnp.zeros((n, ROW - DIM), node_attr.dtype)], axis=1)

    T = jnp.asarray(np.kron(np.eye(DIM), np.ones((1, m))), jnp.float32)
    S = jnp.asarray(np.kron(np.ones((m, 1)), np.eye(m)), jnp.float32)
    cvec = np.full((4 * m * m,), 1.0 / (np.sqrt(2.0 * m) * 16.0), np.float32)
    cvec[3 * m * m:] /= np.sqrt(3.0)
    cvec = jnp.asarray(cvec)
    W2 = W2 * cvec
    b2 = b2 * cvec
    Pm = np.zeros((ROW, DIM), np.float32)
    for u in range(m):
        Pm[u, u] = 1.0
    for k in range(3):
        for w_ in range(m):
            Pm[m + m * k + w_, m + 3 * w_ + k] = 1.0
    P = jnp.asarray(Pm)

    xg = _sc_gather(na, dst)
    tp = _tc_dense(xg, edge_attr.T, edge_feat.T, W1, b1.reshape(1, -1),
                   W2, b2.reshape(1, -1), T, S, block_e=6400)
    parts = _sc_scatter_add(tp, src, n)
    return _tc_combine(parts, P)


# speedup vs baseline: 1.7551x; 1.1199x over previous
import functools

import jax
import jax.numpy as jnp
import numpy as np
from jax import lax
from jax.experimental import pallas as pl
from jax.experimental.pallas import tpu as pltpu
import jax.experimental.pallas.tpu_sc as plsc

MULS = 16
DIM = 4 * MULS
ROW = 128
NC, NS = 2, 16
GW = 128


def _sc_mesh():
    return plsc.VectorSubcoreMesh(
        core_axis_name="core", subcore_axis_name="subcore",
        num_cores=NC, num_subcores=NS)


def _sc_gather(table, idx):
    e = idx.shape[0]
    idx2 = idx.reshape(1, e)

    @functools.partial(
        pl.kernel,
        out_type=jax.ShapeDtypeStruct((e, ROW), jnp.float32),
        mesh=_sc_mesh())
    def k(tab_hbm, i_hbm, o_hbm):
        def body(i_vmem, o_vmem):
            pltpu.sync_copy(tab_hbm.at[i_vmem.at[0]], o_vmem)

        pltpu.emit_pipeline(
            body,
            grid=(e // GW,),
            in_specs=[pl.BlockSpec((1, GW), lambda i: (0, i))],
            out_specs=[pl.BlockSpec((GW, ROW), lambda i: (i, 0))],
            core_axis_name=("core", "subcore"),
            dimension_semantics=(pltpu.PARALLEL,),
        )(i_hbm, o_hbm)

    return k(table, idx2)


def _sc_scatter_add(rows0, rows1, idx, n):
    e = idx.shape[0]
    eh = rows0.shape[0]
    idx2 = idx.reshape(1, e)
    zrows = jnp.zeros((n, ROW), jnp.float32)
    chunk = -(-n // NS) & ~7
    chunk = chunk + 8 if chunk * NS < n else chunk
    rem = n - chunk * (NS - 1)
    assert 0 < rem <= chunk and chunk % 8 == 0

    @functools.partial(
        pl.kernel,
        out_type=jax.ShapeDtypeStruct((NC, n, ROW), jnp.float32),
        mesh=_sc_mesh(),
        scratch_types=[pltpu.VMEM_SHARED((n, ROW), jnp.float32)])
    def k(x0_hbm, x1_hbm, i_hbm, z_hbm, o_hbm, acc):
        cid = lax.axis_index("core")
        sid = lax.axis_index("subcore")

        @pl.when(sid < NS - 1)
        def _():
            sl = pl.ds(sid * chunk, chunk)
            pltpu.sync_copy(z_hbm.at[sl], acc.at[sl])

        @pl.when(sid == NS - 1)
        def _():
            sl = pl.ds((NS - 1) * chunk, rem)
            pltpu.sync_copy(z_hbm.at[sl], acc.at[sl])

        plsc.subcore_barrier()

        def body(x_vmem, i_vmem):
            pltpu.sync_copy(x_vmem, acc.at[i_vmem.at[0]], add=True)

        pltpu.emit_pipeline(
            body,
            grid=(eh // GW,),
            in_specs=[pl.BlockSpec((GW, ROW), lambda i: (i, 0)),
                      pl.BlockSpec((1, GW), lambda i: (0, i))],
            core_axis_name=("core", "subcore"),
            dimension_semantics=(pltpu.PARALLEL,),
        )(x0_hbm, i_hbm)
        pltpu.emit_pipeline(
            body,
            grid=((e - eh) // GW,),
            in_specs=[pl.BlockSpec((GW, ROW), lambda i: (i, 0)),
                      pl.BlockSpec((1, GW),
                                   lambda i: (0, i + eh // GW))],
            core_axis_name=("core", "subcore"),
            dimension_semantics=(pltpu.PARALLEL,),
        )(x1_hbm, i_hbm)
        plsc.subcore_barrier()

        @pl.when(sid < NS - 1)
        def _():
            sl = pl.ds(sid * chunk, chunk)
            pltpu.sync_copy(acc.at[sl], o_hbm.at[cid, sl])

        @pl.when(sid == NS - 1)
        def _():
            sl = pl.ds((NS - 1) * chunk, rem)
            pltpu.sync_copy(acc.at[sl], o_hbm.at[cid, sl])

    return k(rows0, rows1, idx2, zrows)


def _tc_dense(xg, ea, ef, W1, b1, W2, b2, T, S, block_e, e_off=0):
    e = xg.shape[0]
    m = MULS
    bf = jnp.bfloat16
    f32 = jnp.float32

    def body(xg_ref, ea_ref, ef_ref, w1_ref, b1_ref, w2_ref, b2_ref,
             t_ref, s_ref, o_ref):
        mmf = functools.partial(jnp.dot, preferred_element_type=f32)

        def mmb(a, b):
            return mmf(a, b).astype(bf)
        ef_b = jnp.transpose(ef_ref[...])
        h = jnp.maximum(mmf(ef_b, w1_ref[...]) + b1_ref[...], 0.0)
        w = mmb(h.astype(bf), w2_ref[...].astype(bf)) + b2_ref[...].astype(bf)
        xg_b = xg_ref[...].astype(bf)
        ea_b = jnp.transpose(ea_ref[...]).astype(bf)
        t = t_ref[...].astype(bf)
        s = s_ref[...].astype(bf)
        shs = ea_b[:, 0:1]
        sv0 = ea_b[:, 1:2]
        sv1 = ea_b[:, 2:3]
        sv2 = ea_b[:, 3:4]
        wp1 = w[:, :m * m]
        wp2 = w[:, m * m:2 * m * m]
        wp3 = w[:, 2 * m * m:3 * m * m]
        wp4 = w[:, 3 * m * m:4 * m * m]

        xt = mmb(xg_b[:, :DIM], t)
        xst = xt[:, :m * m]
        xv0t = xt[:, m * m:2 * m * m]
        xv1t = xt[:, 2 * m * m:3 * m * m]
        xv2t = xt[:, 3 * m * m:4 * m * m]
        a4 = xv0t * sv0 + xv1t * sv1 + xv2t * sv2
        p2 = wp2 * xst
        outs = mmf(wp1 * xst * shs + wp4 * a4, s)
        ov0 = mmf(p2 * sv0 + wp3 * xv0t * shs, s)
        ov1 = mmf(p2 * sv1 + wp3 * xv1t * shs, s)
        ov2 = mmf(p2 * sv2 + wp3 * xv2t * shs, s)
        pad = jnp.zeros((xg_b.shape[0], ROW - DIM), f32)
        o_ref[...] = jnp.concatenate([outs, ov0, ov1, ov2, pad], axis=1)

    grid = (e // block_e,)
    ob = e_off // block_e
    return pl.pallas_call(
        body,
        grid=grid,
        in_specs=[
            pl.BlockSpec((block_e, ROW), lambda i: (i, 0)),
            pl.BlockSpec((4, block_e), lambda i: (0, i + ob)),
            pl.BlockSpec((ef.shape[0], block_e), lambda i: (0, i + ob)),
            pl.BlockSpec(W1.shape, lambda i: (0, 0)),
            pl.BlockSpec(b1.shape, lambda i: (0, 0)),
            pl.BlockSpec(W2.shape, lambda i: (0, 0)),
            pl.BlockSpec(b2.shape, lambda i: (0, 0)),
            pl.BlockSpec(T.shape, lambda i: (0, 0)),
            pl.BlockSpec(S.shape, lambda i: (0, 0)),
        ],
        out_specs=pl.BlockSpec((block_e, ROW), lambda i: (i, 0)),
        out_shape=jax.ShapeDtypeStruct((e, ROW), jnp.float32),
    )(xg, ea, ef, W1, b1, W2, b2, T, S)


def _tc_combine(parts, P):
    n = parts.shape[1]

    def body(p_ref, perm_ref, o_ref):
        o_ref[...] = jnp.dot(p_ref[0] + p_ref[1], perm_ref[...],
                             preferred_element_type=jnp.float32)

    return pl.pallas_call(
        body,
        out_shape=jax.ShapeDtypeStruct((n, DIM), jnp.float32),
    )(parts, P)


def kernel(node_attr, edge_index, edge_attr, edge_feat, W1, b1, W2, b2):
    n, m = node_attr.shape[0], MULS
    src = edge_index[0]
    dst = edge_index[1]

    na = jnp.pad(node_attr, ((0, 0), (0, ROW - DIM)))

    Tm = np.zeros((DIM, m * DIM), np.float32)
    for c in range(DIM):
        g = c if c < m else m + m * ((c - m) % 3) + (c - m) // 3
        Tm[c, g * m:(g + 1) * m] = 1.0
    T = jnp.asarray(Tm)
    S = jnp.asarray(np.kron(np.ones((m, 1)), np.eye(m)), jnp.float32)
    cvec = np.full((4 * m * m,), 1.0 / (np.sqrt(2.0 * m) * 16.0), np.float32)
    cvec[3 * m * m:] /= np.sqrt(3.0)
    cvec = jnp.asarray(cvec)
    W2 = W2 * cvec
    b2 = b2 * cvec
    Pm = np.zeros((ROW, DIM), np.float32)
    for u in range(m):
        Pm[u, u] = 1.0
    for k in range(3):
        for w_ in range(m):
            Pm[m + m * k + w_, m + 3 * w_ + k] = 1.0
    P = jnp.asarray(Pm)

    e = edge_index.shape[1]
    eh = e // 2
    eaT = edge_attr.T
    efT = edge_feat.T
    b1r = b1.reshape(1, -1)
    b2r = b2.reshape(1, -1)
    xg0 = _sc_gather(na, dst[:eh])
    xg1 = _sc_gather(na, dst[eh:])
    tp0 = _tc_dense(xg0, eaT, efT, W1, b1r, W2, b2r,
                    T, S, block_e=3200)
    tp1 = _tc_dense(xg1, eaT, efT, W1, b1r, W2, b2r,
                    T, S, block_e=3200, e_off=eh)
    parts = _sc_scatter_add(tp0, tp1, src, n)
    return _tc_combine(parts, P)


# speedup vs baseline: 1.8328x; 1.0443x over previous
import functools

import jax
import jax.numpy as jnp
import numpy as np
from jax import lax
from jax.experimental import pallas as pl
from jax.experimental.pallas import tpu as pltpu
import jax.experimental.pallas.tpu_sc as plsc

MULS = 16
DIM = 4 * MULS
ROW = 128
NC, NS = 2, 16
GW = 128


def _sc_mesh():
    return plsc.VectorSubcoreMesh(
        core_axis_name="core", subcore_axis_name="subcore",
        num_cores=NC, num_subcores=NS)


def _sc_gather(table, idx):
    e = idx.shape[0]
    idx2 = idx.reshape(1, e)

    @functools.partial(
        pl.kernel,
        out_type=jax.ShapeDtypeStruct((e, ROW), jnp.float32),
        mesh=_sc_mesh())
    def k(tab_hbm, i_hbm, o_hbm):
        def body(i_vmem, o_vmem):
            pltpu.sync_copy(tab_hbm.at[i_vmem.at[0]], o_vmem)

        pltpu.emit_pipeline(
            body,
            grid=(e // GW,),
            in_specs=[pl.BlockSpec((1, GW), lambda i: (0, i))],
            out_specs=[pl.BlockSpec((GW, ROW), lambda i: (i, 0))],
            core_axis_name=("core", "subcore"),
            dimension_semantics=(pltpu.PARALLEL,),
        )(i_hbm, o_hbm)

    return k(table, idx2)


def _sc_scatter_add(rows, idx, n):
    e = idx.shape[0]
    idx2 = idx.reshape(1, e)
    zrows = jnp.zeros((n, ROW), jnp.float32)
    chunk = -(-n // NS) & ~7
    chunk = chunk + 8 if chunk * NS < n else chunk
    rem = n - chunk * (NS - 1)
    assert 0 < rem <= chunk and chunk % 8 == 0

    @functools.partial(
        pl.kernel,
        out_type=jax.ShapeDtypeStruct((NC, n, ROW), jnp.float32),
        mesh=_sc_mesh(),
        scratch_types=[pltpu.VMEM_SHARED((n, ROW), jnp.float32)])
    def k(x_hbm, i_hbm, z_hbm, o_hbm, acc):
        cid = lax.axis_index("core")
        sid = lax.axis_index("subcore")

        @pl.when(sid < NS - 1)
        def _():
            sl = pl.ds(sid * chunk, chunk)
            pltpu.sync_copy(z_hbm.at[sl], acc.at[sl])

        @pl.when(sid == NS - 1)
        def _():
            sl = pl.ds((NS - 1) * chunk, rem)
            pltpu.sync_copy(z_hbm.at[sl], acc.at[sl])

        plsc.subcore_barrier()

        def body(x_vmem, i_vmem):
            pltpu.sync_copy(x_vmem, acc.at[i_vmem.at[0]], add=True)

        pltpu.emit_pipeline(
            body,
            grid=(e // GW,),
            in_specs=[pl.BlockSpec((GW, ROW), lambda i: (i, 0)),
                      pl.BlockSpec((1, GW), lambda i: (0, i))],
            core_axis_name=("core", "subcore"),
            dimension_semantics=(pltpu.PARALLEL,),
        )(x_hbm, i_hbm)
        plsc.subcore_barrier()

        @pl.when(sid < NS - 1)
        def _():
            sl = pl.ds(sid * chunk, chunk)
            pltpu.sync_copy(acc.at[sl], o_hbm.at[cid, sl])

        @pl.when(sid == NS - 1)
        def _():
            sl = pl.ds((NS - 1) * chunk, rem)
            pltpu.sync_copy(acc.at[sl], o_hbm.at[cid, sl])

    return k(rows, idx2, zrows)


def _tc_dense(xg, ea, ef, W1, b1, W2, b2, T, S, block_e, e_off=0):
    e = xg.shape[0]
    m = MULS
    bf = jnp.bfloat16
    f32 = jnp.float32

    def body(xg_ref, ea_ref, ef_ref, w1_ref, b1_ref, w2_ref, b2_ref,
             t_ref, s_ref, o_ref):
        mmf = functools.partial(jnp.dot, preferred_element_type=f32)

        def mmb(a, b):
            return mmf(a, b).astype(bf)
        ef_b = jnp.transpose(ef_ref[...])
        h = jnp.maximum(mmf(ef_b, w1_ref[...]) + b1_ref[...], 0.0)
        w = mmb(h.astype(bf), w2_ref[...].astype(bf)) + b2_ref[...].astype(bf)
        xg_b = xg_ref[...].astype(bf)
        ea_b = jnp.transpose(ea_ref[...]).astype(bf)
        t = t_ref[...].astype(bf)
        s = s_ref[...].astype(bf)
        shs = ea_b[:, 0:1]
        sv0 = ea_b[:, 1:2]
        sv1 = ea_b[:, 2:3]
        sv2 = ea_b[:, 3:4]
        wp1 = w[:, :m * m]
        wp2 = w[:, m * m:2 * m * m]
        wp3 = w[:, 2 * m * m:3 * m * m]
        wp4 = w[:, 3 * m * m:4 * m * m]

        xt = mmb(xg_b[:, :DIM], t)
        xst = xt[:, :m * m]
        xv0t = xt[:, m * m:2 * m * m]
        xv1t = xt[:, 2 * m * m:3 * m * m]
        xv2t = xt[:, 3 * m * m:4 * m * m]
        a4 = xv0t * sv0 + xv1t * sv1 + xv2t * sv2
        p2 = wp2 * xst
        outs = mmf(wp1 * xst * shs + wp4 * a4, s)
        ov0 = mmf(p2 * sv0 + wp3 * xv0t * shs, s)
        ov1 = mmf(p2 * sv1 + wp3 * xv1t * shs, s)
        ov2 = mmf(p2 * sv2 + wp3 * xv2t * shs, s)
        pad = jnp.zeros((xg_b.shape[0], ROW - DIM), f32)
        o_ref[...] = jnp.concatenate([outs, ov0, ov1, ov2, pad], axis=1)

    grid = (e // block_e,)
    ob = e_off // block_e
    return pl.pallas_call(
        body,
        grid=grid,
        in_specs=[
            pl.BlockSpec((block_e, ROW), lambda i: (i, 0)),
            pl.BlockSpec((4, block_e), lambda i: (0, i + ob)),
            pl.BlockSpec((ef.shape[0], block_e), lambda i: (0, i + ob)),
            pl.BlockSpec(W1.shape, lambda i: (0, 0)),
            pl.BlockSpec(b1.shape, lambda i: (0, 0)),
            pl.BlockSpec(W2.shape, lambda i: (0, 0)),
            pl.BlockSpec(b2.shape, lambda i: (0, 0)),
            pl.BlockSpec(T.shape, lambda i: (0, 0)),
            pl.BlockSpec(S.shape, lambda i: (0, 0)),
        ],
        out_specs=pl.BlockSpec((block_e, ROW), lambda i: (i, 0)),
        out_shape=jax.ShapeDtypeStruct((e, ROW), jnp.float32),
    )(xg, ea, ef, W1, b1, W2, b2, T, S)


def _tc_combine(parts0, parts1, P):
    n = parts0.shape[1]

    def body(p0_ref, p1_ref, perm_ref, o_ref):
        o_ref[...] = jnp.dot(
            (p0_ref[0] + p0_ref[1]) + (p1_ref[0] + p1_ref[1]),
            perm_ref[...], preferred_element_type=jnp.float32)

    return pl.pallas_call(
        body,
        out_shape=jax.ShapeDtypeStruct((n, DIM), jnp.float32),
    )(parts0, parts1, P)


def kernel(node_attr, edge_index, edge_attr, edge_feat, W1, b1, W2, b2):
    n, m = node_attr.shape[0], MULS
    src = edge_index[0]
    dst = edge_index[1]

    na = jnp.pad(node_attr, ((0, 0), (0, ROW - DIM)))

    Tm = np.zeros((DIM, m * DIM), np.float32)
    for c in range(DIM):
        g = c if c < m else m + m * ((c - m) % 3) + (c - m) // 3
        Tm[c, g * m:(g + 1) * m] = 1.0
    T = jnp.asarray(Tm)
    S = jnp.asarray(np.kron(np.ones((m, 1)), np.eye(m)), jnp.float32)
    cvec = np.full((4 * m * m,), 1.0 / (np.sqrt(2.0 * m) * 16.0), np.float32)
    cvec[3 * m * m:] /= np.sqrt(3.0)
    cvec = jnp.asarray(cvec)
    W2 = W2 * cvec
    b2 = b2 * cvec
    Pm = np.zeros((ROW, DIM), np.float32)
    for u in range(m):
        Pm[u, u] = 1.0
    for k in range(3):
        for w_ in range(m):
            Pm[m + m * k + w_, m + 3 * w_ + k] = 1.0
    P = jnp.asarray(Pm)

    e = edge_index.shape[1]
    eh = e // 2
    eaT = edge_attr.T
    efT = edge_feat.T
    b1r = b1.reshape(1, -1)
    b2r = b2.reshape(1, -1)
    xg0 = _sc_gather(na, dst[:eh])
    xg1 = _sc_gather(na, dst[eh:])
    tp0 = _tc_dense(xg0, eaT, efT, W1, b1r, W2, b2r,
                    T, S, block_e=3200)
    tp1 = _tc_dense(xg1, eaT, efT, W1, b1r, W2, b2r,
                    T, S, block_e=3200, e_off=eh)
    parts0 = _sc_scatter_add(tp0, src[:eh], n)
    parts1 = _sc_scatter_add(tp1, src[eh:], n)
    return _tc_combine(parts0, parts1, P)
